# baseline (speedup 1.0000x reference)
import functools

import numpy as np

import jax
import jax.numpy as jnp
from jax.experimental import pallas as pl
from jax.experimental.pallas import tpu as pltpu
from jax.sharding import Mesh, PartitionSpec as P

B, S, D = 1, 2048, 1024
H, K = 16, 8
HD = D // K
HW = H * HD
ROPE_BASE = 10000.0
SB = 256
NB = S // SB
NEG = -1e30
SCALE = HD ** -0.5


def _rope(t, cosf, sa, sb):
    left = jnp.roll(t, -HD // 2, axis=1)
    right = jnp.roll(left, HD, axis=1)
    return t * cosf + left * sa + right * sb


def _qkv_router_body(hl, x_ref, wq_ref, wk_ref, wv_ref, wr_ref, cos_ref,
                     sa_ref, sb_ref, q_ref, k_ref, v_ref, g_ref, stats_ref):
    i = pl.program_id(0)
    x = x_ref[...]
    xb = x.astype(jnp.bfloat16)

    tile = lambda a: jnp.concatenate([a] * hl, axis=1)
    cosf = tile(cos_ref[...])
    sa = tile(sa_ref[...])
    sb = tile(sb_ref[...])
    dn = (((1,), (1,)), ((), ()))
    q = jax.lax.dot_general(xb, wq_ref[...], dn,
                            preferred_element_type=jnp.float32)
    q_ref[...] = (_rope(q, cosf, sa, sb) * SCALE).astype(jnp.bfloat16)
    k = jax.lax.dot_general(xb, wk_ref[...], dn,
                            preferred_element_type=jnp.float32)
    k_ref[...] = _rope(k, cosf, sa, sb).astype(jnp.bfloat16)
    v = jax.lax.dot_general(xb, wv_ref[...], dn,
                            preferred_element_type=jnp.float32)
    v_ref[...] = v.astype(jnp.bfloat16)

    logits = jax.lax.dot_general(xb, wr_ref[...], dn,
                                 preferred_element_type=jnp.float32)

    lo = logits[:, :, None]
    lp = logits[:, None, :]
    hio = jax.lax.broadcasted_iota(jnp.int32, (SB, H, H), 1)
    pio = jax.lax.broadcasted_iota(jnp.int32, (SB, H, H), 2)
    beats = (lp > lo) | ((lp == lo) & (pio < hio))
    rank = jnp.sum(beats.astype(jnp.float32), axis=2)
    sel = rank < float(K)

    m = jnp.max(logits, axis=1, keepdims=True)
    e = jnp.exp(logits - m)
    esel = jnp.where(sel, e, 0.0)
    z_sel = jnp.sum(esel, axis=1, keepdims=True)
    w = esel / z_sel

    hio2 = jax.lax.broadcasted_iota(jnp.int32, (H, H * K), 0)
    jio2 = jax.lax.broadcasted_iota(jnp.int32, (H, H * K), 1)
    rep = ((jio2 // K) == hio2).astype(jnp.float32)
    dn2 = (((1,), (0,)), ((), ()))
    w_b = jax.lax.dot_general(w, rep, dn2, preferred_element_type=jnp.float32)
    rank_b = jax.lax.dot_general(rank, rep, dn2,
                                 preferred_element_type=jnp.float32)
    slot = (jax.lax.broadcasted_iota(jnp.int32, (SB, H * K), 1) % K
            ).astype(jnp.float32)
    g_ref[...] = jnp.where(rank_b == slot, w_b, 0.0)

    @pl.when(i == 0)
    def _init():
        stats_ref[...] = jnp.zeros((8, 128), jnp.float32)

    counts_p = jnp.sum(sel.astype(jnp.float32), axis=0, keepdims=True)
    f_p = jnp.sum((rank == 0.0).astype(jnp.float32), axis=0, keepdims=True)
    z_all = jnp.sum(e, axis=1, keepdims=True)
    p_all = e / z_all
    p_p = jnp.sum(p_all, axis=0, keepdims=True)
    ent = -jnp.sum(p_all * jnp.log(p_all + 1e-08), axis=1, keepdims=True)
    ent_p = jnp.sum(ent, axis=0, keepdims=True)
    lse = m + jnp.log(z_all)
    zl_p = jnp.sum(lse * lse, axis=0, keepdims=True)

    stats_ref[0:1, 0:H] += counts_p
    stats_ref[1:2, 0:H] += f_p
    stats_ref[2:3, 0:H] += p_p
    stats_ref[3:4, 0:1] += ent_p
    stats_ref[4:5, 0:1] += zl_p

    @pl.when(i == NB - 1)
    def _finalize():
        f = stats_ref[1:2, 0:H] / float(S)
        p = stats_ref[2:3, 0:H] / float(S)
        bal = float(H) * jnp.sum(f * p, axis=1, keepdims=True)
        ent_mean = stats_ref[3:4, 0:1] / float(S)
        z_mean = stats_ref[4:5, 0:1] / float(S)
        stats_ref[5:6, 0:1] = 0.01 * bal + 0.01 * (-ent_mean) + 0.01 * z_mean


def _attn_body(hl, q_ref, k_ref, v_ref, g_ref, wo_ref, o_ref):
    i = pl.program_id(0)
    dn_t = (((1,), (1,)), ((), ()))
    dn_n = (((1,), (0,)), ((), ()))
    row = jax.lax.broadcasted_iota(jnp.int32, (SB, SB), 0)
    col = jax.lax.broadcasted_iota(jnp.int32, (SB, SB), 1)
    causal = row >= col
    qs = [q_ref[:, h * HD:(h + 1) * HD] for h in range(hl)]

    ms, ls, accs = [], [], []
    for h in range(hl):
        kb = k_ref[pl.ds(i * SB, SB), h * HD:(h + 1) * HD]
        s = jax.lax.dot_general(qs[h], kb, dn_t,
                                preferred_element_type=jnp.float32)
        s = jnp.where(causal, s, NEG)
        m = jnp.max(s, axis=1, keepdims=True)
        p = jnp.exp(s - m)
        l = jnp.sum(p, axis=1, keepdims=True)
        vb = v_ref[pl.ds(i * SB, SB), h * HD:(h + 1) * HD]
        acc = jax.lax.dot_general(p.astype(jnp.bfloat16), vb, dn_n,
                                  preferred_element_type=jnp.float32)
        ms.append(m)
        ls.append(l)
        accs.append(acc)

    def body(j, carry):
        cms, cls, caccs = carry
        nms, nls, naccs = [], [], []
        for h in range(hl):
            kb = k_ref[pl.ds(j * SB, SB), h * HD:(h + 1) * HD]
            s = jax.lax.dot_general(qs[h], kb, dn_t,
                                    preferred_element_type=jnp.float32)
            m2 = jnp.max(s, axis=1, keepdims=True)
            mn = jnp.maximum(cms[h], m2)
            p = jnp.exp(s - mn)
            alpha = jnp.exp(cms[h] - mn)
            vb = v_ref[pl.ds(j * SB, SB), h * HD:(h + 1) * HD]
            pv = jax.lax.dot_general(p.astype(jnp.bfloat16), vb, dn_n,
                                     preferred_element_type=jnp.float32)
            nms.append(mn)
            nls.append(cls[h] * alpha + jnp.sum(p, axis=1, keepdims=True))
            naccs.append(caccs[h] * alpha + pv)
        return tuple(nms), tuple(nls), tuple(naccs)

    ms, ls, accs = jax.lax.fori_loop(
        0, i, body, (tuple(ms), tuple(ls), tuple(accs)))

    parts = [jnp.zeros((SB, HD), jnp.bfloat16) for _ in range(K)]
    for h in range(hl):
        oh = (accs[h] / ls[h]).astype(jnp.bfloat16)
        gh = g_ref[:, h * K:(h + 1) * K].astype(jnp.bfloat16)
        for kk in range(K):
            parts[kk] = parts[kk] + gh[:, kk:kk + 1] * oh
    ctx = jnp.concatenate(parts, axis=1)
    o_ref[...] = jax.lax.dot_general(ctx, wo_ref[...], dn_t,
                                     preferred_element_type=jnp.float32)


def _tables():
    inv_freq = 1.0 / (ROPE_BASE ** (jnp.arange(0, HD, 2, dtype=jnp.float32) / HD))
    t = jnp.arange(S, dtype=jnp.float32)
    freqs = jnp.outer(t, inv_freq)
    emb = jnp.concatenate([freqs, freqs], axis=-1)
    cos = jnp.cos(emb)
    sin = jnp.sin(emb)
    half = jnp.arange(HD) < (HD // 2)
    sa = jnp.where(half[None, :], -sin, 0.0)
    sb = jnp.where(half[None, :], 0.0, sin)
    return cos, sa, sb


def _device_fn(nd, x2, wq_b, wk_b, wv_b, wr_b, wo_b, cosf, sa, sb,
               interpret=False):
    hl = H // nd
    hw = hl * HD

    const = lambda i: (0, 0)
    blk = lambda i: (i, 0)
    q, k, v, g, stats = pl.pallas_call(
        functools.partial(_qkv_router_body, hl),
        grid=(NB,),
        in_specs=[
            pl.BlockSpec((SB, D), blk),
            pl.BlockSpec((hw, D), const),
            pl.BlockSpec((hw, D), const),
            pl.BlockSpec((hw, D), const),
            pl.BlockSpec((H, D), const),
            pl.BlockSpec((SB, HD), blk),
            pl.BlockSpec((SB, HD), blk),
            pl.BlockSpec((SB, HD), blk),
        ],
        out_specs=[
            pl.BlockSpec((SB, hw), blk),
            pl.BlockSpec((SB, hw), blk),
            pl.BlockSpec((SB, hw), blk),
            pl.BlockSpec((SB, H * K), blk),
            pl.BlockSpec((8, 128), const),
        ],
        out_shape=[
            jax.ShapeDtypeStruct((S, hw), jnp.bfloat16),
            jax.ShapeDtypeStruct((S, hw), jnp.bfloat16),
            jax.ShapeDtypeStruct((S, hw), jnp.bfloat16),
            jax.ShapeDtypeStruct((S, H * K), jnp.float32),
            jax.ShapeDtypeStruct((8, 128), jnp.float32),
        ],
        interpret=interpret,
    )(x2, wq_b, wk_b, wv_b, wr_b, cosf, sa, sb)

    if nd > 1:
        d = jax.lax.axis_index('d')
        g_loc = jax.lax.dynamic_slice(g, (0, d * hl * K), (S, hl * K))
    else:
        g_loc = g

    out = pl.pallas_call(
        functools.partial(_attn_body, hl),
        grid=(NB,),
        in_specs=[
            pl.BlockSpec((SB, hw), blk),
            pl.BlockSpec((S, hw), const),
            pl.BlockSpec((S, hw), const),
            pl.BlockSpec((SB, hl * K), blk),
            pl.BlockSpec((D, D), const),
        ],
        out_specs=pl.BlockSpec((SB, D), blk),
        out_shape=jax.ShapeDtypeStruct((S, D), jnp.float32),
        interpret=interpret,
    )(q, k, v, g_loc, wo_b)

    if nd > 1:
        out = jax.lax.psum(out, 'd')
    return out, stats


def _run(x2, wq, wk, wv, wr, wo, cosf, sa, sb, interpret=False):
    wq_b = wq.astype(jnp.bfloat16)
    wk_b = wk.astype(jnp.bfloat16)
    wv_b = wv.astype(jnp.bfloat16)
    wr_b = wr.astype(jnp.bfloat16)
    wo_b = wo.astype(jnp.bfloat16)

    devs = jax.devices()
    nd = 2 if len(devs) >= 2 else 1
    if nd == 1:
        return _device_fn(1, x2, wq_b, wk_b, wv_b, wr_b, wo_b, cosf, sa, sb,
                          interpret=interpret)

    mesh = Mesh(np.array(devs[:nd]), ('d',))
    fn = jax.shard_map(
        functools.partial(_device_fn, nd, interpret=interpret),
        mesh=mesh,
        in_specs=(P(None, None), P('d', None), P('d', None), P('d', None),
                  P(None, None), P(None, None), P(None, None), P(None, None),
                  P(None, None)),
        out_specs=(P(None, None), P(None, None)),
        check_vma=False,
    )
    return fn(x2, wq_b, wk_b, wv_b, wr_b, wo_b, cosf, sa, sb)


def kernel(x, Wq, Wk, Wv, Wr, Wo):
    x2 = x.reshape(S, D)
    cosf, sa, sb = _tables()
    out, stats = _run(x2, Wq, Wk, Wv, Wr, Wo, cosf, sa, sb)
    counts = stats[0:1, 0:H].astype(jnp.int32)
    aux = stats[5, 0]
    return out.reshape(B, S, D), counts, aux


# speedup vs baseline: 2.7285x; 2.7285x over previous
import functools

import jax
import jax.numpy as jnp
from jax import lax
from jax.experimental import pallas as pl
from jax.experimental.pallas import tpu as pltpu
from jax.experimental.pallas import tpu_sc as plsc

B, S, D = 1, 2048, 1024
H, K = 16, 8
HD = D // K
HW = H * HD
ROPE_BASE = 10000.0
SB = 256
NB = S // SB
NEG = -1e30
SCALE = HD ** -0.5


def _rope(t, cosf, sa, sb):
    left = jnp.roll(t, -HD // 2, axis=1)
    right = jnp.roll(left, HD, axis=1)
    return t * cosf + left * sa + right * sb


def _logits_body(x_ref, wr_ref, lg_ref, stats_ref):
    i = pl.program_id(0)
    xb = x_ref[...].astype(jnp.bfloat16)
    dn = (((1,), (1,)), ((), ()))
    logits = jax.lax.dot_general(xb, wr_ref[...], dn,
                                 preferred_element_type=jnp.float32)
    lg_ref[...] = logits

    @pl.when(i == 0)
    def _init():
        stats_ref[...] = jnp.zeros((8, 128), jnp.float32)

    m = jnp.max(logits, axis=1, keepdims=True)
    e = jnp.exp(logits - m)
    z_all = jnp.sum(e, axis=1, keepdims=True)
    p_all = e / z_all
    p_p = jnp.sum(p_all, axis=0, keepdims=True)
    ent = -jnp.sum(p_all * jnp.log(p_all + 1e-08), axis=1, keepdims=True)
    ent_p = jnp.sum(ent, axis=0, keepdims=True)
    lse = m + jnp.log(z_all)
    zl_p = jnp.sum(lse * lse, axis=0, keepdims=True)
    stats_ref[2:3, 0:H] += p_p
    stats_ref[3:4, 0:1] += ent_p
    stats_ref[4:5, 0:1] += zl_p


def _sc_routing(logits):
    info = plsc.get_sparse_core_info()
    nw = info.num_cores * info.num_subcores
    tpw = S // nw
    mesh = plsc.VectorSubcoreMesh(core_axis_name="c", subcore_axis_name="s")

    @functools.partial(
        pl.kernel,
        mesh=mesh,
        out_type=[
            jax.ShapeDtypeStruct((S * H * K,), jnp.float32),
            jax.ShapeDtypeStruct((nw, H), jnp.int32),
            jax.ShapeDtypeStruct((nw, H), jnp.int32),
        ],
        scratch_types=[
            pltpu.VMEM((tpw, H), jnp.float32),
            pltpu.VMEM((tpw * H * K,), jnp.float32),
            pltpu.VMEM((H,), jnp.int32),
            pltpu.VMEM((H,), jnp.int32),
            pltpu.VMEM((2 * H,), jnp.float32),
            pltpu.VMEM((2 * H,), jnp.float32),
        ],
    )
    def route(lg_hbm, g_hbm, cnt_hbm, fc_hbm, l_v, g_v, cnt_v, fc_v,
              dbl_v, bfy_v):
        wid = lax.axis_index("s") * info.num_cores + lax.axis_index("c")
        base = wid * tpw
        pltpu.sync_copy(lg_hbm.at[pl.ds(base, tpw), :], l_v)
        lanes = lax.iota(jnp.int32, 16)
        zeros16 = jnp.zeros((16,), jnp.float32)
        zerosi = jnp.zeros((16,), jnp.int32)
        onesi = jnp.ones((16,), jnp.int32)
        cnt_v[...] = zerosi
        fc_v[...] = zerosi

        def bfly(x, op):
            cur = x
            for st in (8, 4, 2, 1):
                bfy_v[pl.ds(0, 16)] = cur
                bfy_v[pl.ds(16, 16)] = cur
                cur = op(cur, bfy_v[pl.ds(st, 16)])
            return cur

        def tok(t, carry):
            l = l_v[t, :]
            dbl_v[pl.ds(0, 16)] = l
            dbl_v[pl.ds(16, 16)] = l
            rank = zerosi
            for dd in range(1, H):
                ld = dbl_v[pl.ds(dd, 16)]
                hp = lax.rem(lanes + dd, H)
                beats = (ld > l) | ((ld == l) & (hp < lanes))
                rank = rank + jnp.where(beats, onesi, zerosi)
            sel = rank < K
            m_b = bfly(l, jnp.maximum)
            e = jnp.exp(l - m_b)
            esel = jnp.where(sel, e, zeros16)
            z_b = bfly(esel, jnp.add)
            w = esel / z_b
            for kb in range(K):
                g_v[pl.ds(t * (H * K) + kb * 16, 16)] = jnp.where(
                    rank == kb, w, zeros16)
            cnt_v[...] = cnt_v[...] + jnp.where(sel, onesi, zerosi)
            fc_v[...] = fc_v[...] + jnp.where(rank == 0, onesi, zerosi)
            return carry

        lax.fori_loop(0, tpw, tok, 0)
        pltpu.sync_copy(g_v, g_hbm.at[pl.ds(base * (H * K), tpw * (H * K))])
        pltpu.sync_copy(cnt_v, cnt_hbm.at[wid])
        pltpu.sync_copy(fc_v, fc_hbm.at[wid])

    return route(logits)


def _qkv_body(x_ref, wq_ref, wk_ref, wv_ref, cos_ref, sa_ref, sb_ref,
              q_ref, k_ref, v_ref):
    x = x_ref[...]
    xb = x.astype(jnp.bfloat16)
    tile = lambda a: jnp.concatenate([a] * H, axis=1)
    cosf = tile(cos_ref[...])
    sa = tile(sa_ref[...])
    sb = tile(sb_ref[...])
    dn = (((1,), (1,)), ((), ()))
    q = jax.lax.dot_general(xb, wq_ref[...], dn,
                            preferred_element_type=jnp.float32)
    q_ref[...] = (_rope(q, cosf, sa, sb) * SCALE).astype(jnp.bfloat16)
    k = jax.lax.dot_general(xb, wk_ref[...], dn,
                            preferred_element_type=jnp.float32)
    k_ref[...] = _rope(k, cosf, sa, sb).astype(jnp.bfloat16)
    v = jax.lax.dot_general(xb, wv_ref[...], dn,
                            preferred_element_type=jnp.float32)
    v_ref[...] = v.astype(jnp.bfloat16)


def _attn_body(q_ref, k_ref, v_ref, g_ref, wo_ref, o_ref):
    i = pl.program_id(0)
    dn_t = (((1,), (1,)), ((), ()))
    dn_n = (((1,), (0,)), ((), ()))
    row = jax.lax.broadcasted_iota(jnp.int32, (SB, SB), 0)
    col = jax.lax.broadcasted_iota(jnp.int32, (SB, SB), 1)
    causal = row >= col
    qs = [q_ref[:, h * HD:(h + 1) * HD] for h in range(H)]

    ms, ls, accs = [], [], []
    for h in range(H):
        kb = k_ref[pl.ds(i * SB, SB), h * HD:(h + 1) * HD]
        s = jax.lax.dot_general(qs[h], kb, dn_t,
                                preferred_element_type=jnp.float32)
        s = jnp.where(causal, s, NEG)
        m = jnp.max(s, axis=1, keepdims=True)
        p = jnp.exp(s - m)
        l = jnp.sum(p, axis=1, keepdims=True)
        vb = v_ref[pl.ds(i * SB, SB), h * HD:(h + 1) * HD]
        acc = jax.lax.dot_general(p.astype(jnp.bfloat16), vb, dn_n,
                                  preferred_element_type=jnp.float32)
        ms.append(m)
        ls.append(l)
        accs.append(acc)

    def body(j, carry):
        cms, cls, caccs = carry
        nms, nls, naccs = [], [], []
        for h in range(H):
            kb = k_ref[pl.ds(j * SB, SB), h * HD:(h + 1) * HD]
            s = jax.lax.dot_general(qs[h], kb, dn_t,
                                    preferred_element_type=jnp.float32)
            m2 = jnp.max(s, axis=1, keepdims=True)
            mn = jnp.maximum(cms[h], m2)
            p = jnp.exp(s - mn)
            alpha = jnp.exp(cms[h] - mn)
            vb = v_ref[pl.ds(j * SB, SB), h * HD:(h + 1) * HD]
            pv = jax.lax.dot_general(p.astype(jnp.bfloat16), vb, dn_n,
                                     preferred_element_type=jnp.float32)
            nms.append(mn)
            nls.append(cls[h] * alpha + jnp.sum(p, axis=1, keepdims=True))
            naccs.append(caccs[h] * alpha + pv)
        return tuple(nms), tuple(nls), tuple(naccs)

    ms, ls, accs = jax.lax.fori_loop(
        0, i, body, (tuple(ms), tuple(ls), tuple(accs)))

    gb = g_ref[...].astype(jnp.bfloat16)
    parts = [jnp.zeros((SB, HD), jnp.bfloat16) for _ in range(K)]
    for h in range(H):
        oh = (accs[h] / ls[h]).astype(jnp.bfloat16)
        for kk in range(K):
            parts[kk] = parts[kk] + gb[:, kk * H + h:kk * H + h + 1] * oh
    ctx = jnp.concatenate(parts, axis=1)
    o_ref[...] = jax.lax.dot_general(ctx, wo_ref[...], dn_t,
                                     preferred_element_type=jnp.float32)


def _tables():
    inv_freq = 1.0 / (ROPE_BASE ** (jnp.arange(0, HD, 2, dtype=jnp.float32) / HD))
    t = jnp.arange(S, dtype=jnp.float32)
    freqs = jnp.outer(t, inv_freq)
    emb = jnp.concatenate([freqs, freqs], axis=-1)
    cos = jnp.cos(emb)
    sin = jnp.sin(emb)
    half = jnp.arange(HD) < (HD // 2)
    sa = jnp.where(half[None, :], -sin, 0.0)
    sb = jnp.where(half[None, :], 0.0, sin)
    return cos, sa, sb


def kernel(x, Wq, Wk, Wv, Wr, Wo):
    x2 = x.reshape(S, D)
    cosf, sa, sb = _tables()
    wq_b = Wq.astype(jnp.bfloat16)
    wk_b = Wk.astype(jnp.bfloat16)
    wv_b = Wv.astype(jnp.bfloat16)
    wr_b = Wr.astype(jnp.bfloat16)
    wo_b = Wo.astype(jnp.bfloat16)

    const = lambda i: (0, 0)
    blk = lambda i: (i, 0)

    lg, stats = pl.pallas_call(
        _logits_body,
        grid=(NB,),
        in_specs=[
            pl.BlockSpec((SB, D), blk),
            pl.BlockSpec((H, D), const),
        ],
        out_specs=[
            pl.BlockSpec((SB, H), blk),
            pl.BlockSpec((8, 128), const),
        ],
        out_shape=[
            jax.ShapeDtypeStruct((S, H), jnp.float32),
            jax.ShapeDtypeStruct((8, 128), jnp.float32),
        ],
    )(x2, wr_b)

    g_flat, cnt, fc = _sc_routing(lg)
    g = g_flat.reshape(S, H * K)

    q, k, v = pl.pallas_call(
        _qkv_body,
        grid=(NB,),
        in_specs=[
            pl.BlockSpec((SB, D), blk),
            pl.BlockSpec((HW, D), const),
            pl.BlockSpec((HW, D), const),
            pl.BlockSpec((HW, D), const),
            pl.BlockSpec((SB, HD), blk),
            pl.BlockSpec((SB, HD), blk),
            pl.BlockSpec((SB, HD), blk),
        ],
        out_specs=[
            pl.BlockSpec((SB, HW), blk),
            pl.BlockSpec((SB, HW), blk),
            pl.BlockSpec((SB, HW), blk),
        ],
        out_shape=[
            jax.ShapeDtypeStruct((S, HW), jnp.bfloat16),
            jax.ShapeDtypeStruct((S, HW), jnp.bfloat16),
            jax.ShapeDtypeStruct((S, HW), jnp.bfloat16),
        ],
    )(x2, wq_b, wk_b, wv_b, cosf, sa, sb)

    out = pl.pallas_call(
        _attn_body,
        grid=(NB,),
        in_specs=[
            pl.BlockSpec((SB, HW), blk),
            pl.BlockSpec((S, HW), const),
            pl.BlockSpec((S, HW), const),
            pl.BlockSpec((SB, H * K), blk),
            pl.BlockSpec((D, D), const),
        ],
        out_specs=pl.BlockSpec((SB, D), blk),
        out_shape=jax.ShapeDtypeStruct((S, D), jnp.float32),
    )(q, k, v, g, wo_b)

    counts = jnp.sum(cnt, axis=0, dtype=jnp.int32).reshape(1, H)
    fbar = jnp.sum(fc, axis=0).astype(jnp.float32) / float(S)
    pbar = stats[2, 0:H] / float(S)
    bal = float(H) * jnp.sum(fbar * pbar)
    ent_mean = stats[3, 0] / float(S)
    z_mean = stats[4, 0] / float(S)
    aux = 0.01 * bal + 0.01 * (-ent_mean) + 0.01 * z_mean
    return out.reshape(B, S, D), counts, aux
